# baseline (speedup 1.0000x reference)
import functools

import jax
import jax.numpy as jnp
from jax import lax
from jax.experimental import pallas as pl
from jax.experimental.pallas import tpu as pltpu
from jax.experimental.pallas import tpu_sc as plsc

SEQ = 200
EMBED = 64
LANES = 16
NUM_WORKERS = 32
CHUNK = 400
GATHER_W = 200
EMBED_SCALE = 8.0
NDIM = EMBED // LANES


def _body(idx_hbm, tok2_hbm, pos2_hbm, out_hbm, idx_v, pidx_v, rows2_v, out_v, pos_v, sem):
    c = lax.axis_index("c")
    s = lax.axis_index("s")
    wid = s * 2 + c
    n_rows = out_hbm.shape[0]
    rows_per_worker = n_rows // NUM_WORKERS
    chunks_per_worker = rows_per_worker // CHUNK

    pltpu.sync_copy(pos2_hbm, pos_v)

    def chunk_body(ci, _):
        base = wid * rows_per_worker + ci * CHUNK

        pltpu.sync_copy(idx_hbm.at[pl.ds(base, CHUNK)], idx_v.at[pl.ds(0, CHUNK)])

        def shift_body(i, _):
            sl = pl.ds(i * LANES, LANES)
            pidx_v[sl] = lax.shift_right_logical(idx_v[sl], 1)
            return _

        lax.fori_loop(0, CHUNK // LANES, shift_body, None)

        copies = []
        for j in range(CHUNK // GATHER_W):
            copies.append(
                pltpu.async_copy(
                    tok2_hbm.at[pidx_v.at[pl.ds(j * GATHER_W, GATHER_W)]],
                    rows2_v.at[pl.ds(j * GATHER_W, GATHER_W)],
                    sem,
                )
            )
        for cp in copies:
            cp.wait()

        def pos_body(q, _):
            for p_off in range(2):
                pv = [
                    pos_v[q, pl.ds(p_off * EMBED + d * LANES, LANES)]
                    for d in range(NDIM)
                ]
                for jb in range(CHUNK // SEQ):
                    r = jb * SEQ + 2 * q + p_off
                    hv = idx_v[pl.ds(r, LANES)]
                    h = (hv[0] & 1) * EMBED
                    for d in range(NDIM):
                        src = rows2_v[r, pl.ds(h + d * LANES, LANES)]
                        out_v[r, pl.ds(d * LANES, LANES)] = (
                            src * EMBED_SCALE + pv[d]
                        )
            return _

        lax.fori_loop(0, SEQ // 2, pos_body, None)

        pltpu.sync_copy(out_v, out_hbm.at[pl.ds(base, CHUNK)])
        return _

    lax.fori_loop(0, chunks_per_worker, chunk_body, None)


def kernel(inputs, token_table, position_table):
    batch, seq = inputs.shape
    vocab = token_table.shape[0]
    n_rows = batch * seq
    idx = jnp.minimum(inputs, vocab - 1).reshape(n_rows)
    tok2 = token_table.reshape(vocab // 2, 2 * EMBED)
    pos2 = position_table.reshape(seq // 2, 2 * EMBED)

    mesh = plsc.VectorSubcoreMesh(core_axis_name="c", subcore_axis_name="s")
    k = functools.partial(
        pl.kernel,
        mesh=mesh,
        out_type=jax.ShapeDtypeStruct((n_rows, EMBED), jnp.float32),
        scratch_types=[
            pltpu.VMEM((CHUNK + LANES,), jnp.int32),
            pltpu.VMEM((CHUNK,), jnp.int32),
            pltpu.VMEM((CHUNK, 2 * EMBED), jnp.float32),
            pltpu.VMEM((CHUNK, EMBED), jnp.float32),
            pltpu.VMEM((seq // 2, 2 * EMBED), jnp.float32),
            pltpu.SemaphoreType.DMA,
        ],
        compiler_params=pltpu.CompilerParams(use_tc_tiling_on_sc=True),
    )(_body)

    out = k(idx, tok2, pos2)
    return out.reshape(batch, seq, EMBED)


# speedup vs baseline: 1.4895x; 1.4895x over previous
import functools

import jax
import jax.numpy as jnp
from jax import lax
from jax.experimental import pallas as pl
from jax.experimental.pallas import tpu as pltpu
from jax.experimental.pallas import tpu_sc as plsc

SEQ = 200
EMBED = 64
LANES = 16
NUM_WORKERS = 32
CHUNK = 400
GATHER_W = 200
EMBED_SCALE = 8.0
NDIM = EMBED // LANES


def _body(idx_hbm, tok2_hbm, pos2_hbm, out_hbm, idx_v, pidx_v, rows2_v, out_v, pos_v, sem):
    c = lax.axis_index("c")
    s = lax.axis_index("s")
    wid = s * 2 + c
    n_rows = out_hbm.shape[0]
    rows_per_worker = n_rows // NUM_WORKERS
    chunks_per_worker = rows_per_worker // CHUNK

    pltpu.sync_copy(pos2_hbm, pos_v)

    def chunk_body(ci, _):
        base = wid * rows_per_worker + ci * CHUNK

        pltpu.sync_copy(idx_hbm.at[pl.ds(base, CHUNK)], idx_v.at[pl.ds(0, CHUNK)])

        def shift_body(i, _):
            sl = pl.ds(i * LANES, LANES)
            pidx_v[sl] = lax.shift_right_logical(idx_v[sl], 1)
            return _

        lax.fori_loop(0, CHUNK // LANES, shift_body, None)

        copies = []
        for j in range(CHUNK // GATHER_W):
            copies.append(
                pltpu.async_copy(
                    tok2_hbm.at[pidx_v.at[pl.ds(j * GATHER_W, GATHER_W)]],
                    rows2_v.at[pl.ds(j * GATHER_W, GATHER_W)],
                    sem,
                )
            )
        for cp in copies:
            cp.wait()

        def pos_body(q, _):
            for p_off in range(2):
                pv = [
                    pos_v[q, pl.ds(p_off * EMBED + d * LANES, LANES)]
                    for d in range(NDIM)
                ]
                for jb in range(CHUNK // SEQ):
                    r = jb * SEQ + 2 * q + p_off
                    rv = plsc.load_gather(
                        idx_v, [jnp.full((LANES,), r, jnp.int32)]
                    )
                    odd = (rv & 1) == 1
                    for d in range(NDIM):
                        lo = rows2_v[r, pl.ds(d * LANES, LANES)]
                        hi = rows2_v[r, pl.ds(EMBED + d * LANES, LANES)]
                        sel = jnp.where(odd, hi, lo)
                        out_v[r, pl.ds(d * LANES, LANES)] = (
                            sel * EMBED_SCALE + pv[d]
                        )
            return _

        lax.fori_loop(0, SEQ // 2, pos_body, None)

        pltpu.sync_copy(out_v, out_hbm.at[pl.ds(base, CHUNK)])
        return _

    lax.fori_loop(0, chunks_per_worker, chunk_body, None)


def kernel(inputs, token_table, position_table):
    batch, seq = inputs.shape
    vocab = token_table.shape[0]
    n_rows = batch * seq
    idx = jnp.minimum(inputs.reshape(n_rows), vocab - 1)
    tok2 = token_table.reshape(vocab // 2, 2 * EMBED)
    pos2 = position_table.reshape(seq // 2, 2 * EMBED)

    mesh = plsc.VectorSubcoreMesh(core_axis_name="c", subcore_axis_name="s")
    k = functools.partial(
        pl.kernel,
        mesh=mesh,
        out_type=jax.ShapeDtypeStruct((n_rows, EMBED), jnp.float32),
        scratch_types=[
            pltpu.VMEM((CHUNK + LANES,), jnp.int32),
            pltpu.VMEM((CHUNK,), jnp.int32),
            pltpu.VMEM((CHUNK, 2 * EMBED), jnp.float32),
            pltpu.VMEM((CHUNK, EMBED), jnp.float32),
            pltpu.VMEM((seq // 2, 2 * EMBED), jnp.float32),
            pltpu.SemaphoreType.DMA,
        ],
        compiler_params=pltpu.CompilerParams(
            use_tc_tiling_on_sc=True, needs_layout_passes=False
        ),
    )(_body)

    out = k(idx, tok2, pos2)
    return out.reshape(batch, seq, EMBED)
